# baseline (speedup 1.0000x reference)
import jax
import jax.numpy as jnp
from jax import lax
from jax.experimental import pallas as pl
from jax.experimental.pallas import tpu as pltpu
from jax.experimental.pallas import tpu_sc as plsc

OBS = 128
DIMS = 8192
LANES = 16
CHUNKS = DIMS // LANES
NC = 2
NS = 16
NW = NC * NS
RPW = OBS // NW
BISECT = 30
UNROLL = 16
TRIPS = CHUNKS // UNROLL


def _zeros():
    return jnp.zeros((LANES,), jnp.float32)


def _treemax(cs):
    cs = list(cs)
    while len(cs) > 1:
        cs = [jnp.maximum(cs[j], cs[j + 1]) for j in range(0, len(cs), 2)]
    return cs[0]


def _pool16(v):
    for sh in (8, 4, 2, 1):
        idx = jnp.bitwise_xor(lax.iota(jnp.int32, LANES), sh)
        v = jnp.maximum(v, v[idx])
    return v


def _sparsemax_body(x_hbm, out_hbm, buf, icand, cand2, icand2, cand3,
                    icand3, obuf, *sems):
    isems = sems[:RPW]
    osems = sems[RPW:]
    wid = lax.axis_index("s") * NC + lax.axis_index("c")
    base = wid * RPW

    in_copies = [
        pltpu.async_copy(x_hbm.at[base + j], buf.at[j], isems[j])
        for j in range(RPW)
    ]

    @plsc.parallel_loop(0, CHUNKS)
    def zfill(i):
        obuf[pl.ds(i * LANES, LANES)] = _zeros()

    prev_iv = lax.iota(jnp.int32, LANES)
    prev_msk = lax.iota(jnp.int32, LANES) < 0

    for r in range(RPW):
        in_copies[r].wait()
        rvec = lax.iota(jnp.int32, LANES) * 0 + r

        def loadtrip(i):
            return [buf[r, pl.ds(i * (UNROLL * LANES) + k * LANES, LANES)]
                    for k in range(UNROLL)]

        def ivecs(i):
            return [lax.iota(jnp.int32, LANES) + (i * (UNROLL * LANES)
                                                  + k * LANES)
                    for k in range(UNROLL)]

        def compact_idx(off, ivs, msks):
            pcs = [plsc.all_reduce_population_count(m)[0] for m in msks]
            for k in range(len(ivs)):
                plsc.store_compressed(icand.at[pl.ds(off, LANES)], ivs[k],
                                      mask=msks[k])
                off = off + pcs[k]
            return off

        first = loadtrip(0)
        gm0 = _pool16(_treemax(first))
        thr0 = gm0 - 1.0
        cnt0 = compact_idx(jnp.int32(0), ivecs(0),
                           [c > thr0 for c in first])

        def fz_body(i, carry):
            gm, cnt = carry
            cs = loadtrip(i)
            thr = gm - 1.0
            cnt = compact_idx(cnt, ivecs(i), [c > thr for c in cs])
            return jnp.maximum(gm, _pool16(_treemax(cs))), cnt

        gm, cnt = lax.fori_loop(1, TRIPS, fz_body, (gm0, cnt0))
        mv = gm
        lo0 = mv - 1.0
        icand[pl.ds(cnt, LANES)] = jnp.zeros((LANES,), jnp.int32)

        def s2_body(i, carry):
            c2, kv, sv = carry
            iv = icand[pl.ds(i * LANES, LANES)]
            c = plsc.load_gather(buf, [rvec, iv])
            msk = c > lo0
            plsc.store_compressed(cand2.at[pl.ds(c2, LANES)], c, mask=msk)
            plsc.store_compressed(icand2.at[pl.ds(c2, LANES)], iv, mask=msk)
            return (c2 + plsc.all_reduce_population_count(msk)[0],
                    kv + jnp.where(msk, 1.0, 0.0),
                    sv + jnp.where(msk, c, 0.0))

        nch1 = lax.shift_right_logical(cnt + (LANES - 1), 4)
        cnt2, kv0, sv0 = lax.fori_loop(0, nch1, s2_body,
                                       (jnp.int32(0), _zeros(), _zeros()))
        cand2[pl.ds(cnt2, LANES)] = lo0
        icand2[pl.ds(cnt2, LANES)] = jnp.zeros((LANES,), jnp.int32)
        nch2 = lax.shift_right_logical(cnt2 + (LANES - 1), 4)

        tm = jnp.maximum((_zeros() + jnp.sum(sv0) - 1.0) /
                         jnp.maximum(_zeros() + jnp.sum(kv0), 1.0), lo0)

        def mich_round(_, tm):
            def mb(i, carry):
                kv, sv = carry
                c = cand2[pl.ds(i * LANES, LANES)]
                msk = c > tm
                return (kv + jnp.where(msk, 1.0, 0.0),
                        sv + jnp.where(msk, c, 0.0))

            kv, sv = lax.fori_loop(0, nch2, mb, (_zeros(), _zeros()))
            return jnp.maximum(
                (_zeros() + jnp.sum(sv) - 1.0) /
                jnp.maximum(_zeros() + jnp.sum(kv), 1.0), tm)

        tm = lax.fori_loop(0, 2, mich_round, tm)

        def s3_body(i, c3):
            c = cand2[pl.ds(i * LANES, LANES)]
            iv = icand2[pl.ds(i * LANES, LANES)]
            msk = c > tm
            plsc.store_compressed(cand3.at[pl.ds(c3, LANES)], c, mask=msk)
            plsc.store_compressed(icand3.at[pl.ds(c3, LANES)], iv, mask=msk)
            return c3 + plsc.all_reduce_population_count(msk)[0]

        cnt3 = lax.fori_loop(0, nch2, s3_body, jnp.int32(0))
        cand3[pl.ds(cnt3, LANES)] = tm
        icand3[pl.ds(cnt3, LANES)] = jnp.zeros((LANES,), jnp.int32)

        def vreg_path(_):
            cv = cand3[pl.ds(0, LANES)]
            sk, _sv = plsc.sort_key_val(cv, cv, descending=True)
            csum = plsc.cumsum(sk)
            kf = (lax.iota(jnp.int32, LANES) + 1).astype(jnp.float32)
            check = 1.0 + kf * sk > csum
            kz = plsc.all_reduce_population_count(check)
            tau_sum = csum[kz - 1]
            return (tau_sum - 1.0) / kz.astype(jnp.float32)

        def bisect_path(_):
            def bis_body(j, carry):
                lo, hi = carry
                t = (lo + hi) * 0.5

                def ps(i, a):
                    c = cand2[pl.ds(i * LANES, LANES)]
                    return a + jnp.maximum(c - t, 0.0)

                sv = _zeros() + jnp.sum(lax.fori_loop(0, nch2, ps, _zeros()))
                big = sv >= 1.0
                return jnp.where(big, t, lo), jnp.where(big, hi, t)

            lo, hi = lax.fori_loop(0, BISECT, bis_body, (lo0, mv))

            def ex_body(i, carry):
                kv, sv = carry
                c = cand2[pl.ds(i * LANES, LANES)]
                msk = c > hi
                return (kv + jnp.where(msk, 1.0, 0.0),
                        sv + jnp.where(msk, c, 0.0))

            kv, sv = lax.fori_loop(0, nch2, ex_body, (_zeros(), _zeros()))
            ks = jnp.maximum(_zeros() + jnp.sum(kv), 1.0)
            ss = _zeros() + jnp.sum(sv)
            return (ss - 1.0) / ks

        take_scatter = cnt3 <= LANES
        t_ex = lax.cond(take_scatter, vreg_path, bisect_path, 0)

        cv3 = cand3[pl.ds(0, LANES)]
        iv3 = icand3[pl.ds(0, LANES)]

        if r > 0:
            pltpu.make_async_copy(obuf, out_hbm.at[base + r - 1],
                                  osems[r - 1]).wait()
            plsc.store_scatter(obuf, [prev_iv], _zeros(), mask=prev_msk)

        taken_f = _zeros() + jnp.where(take_scatter, 1.0, 0.0)
        sup_msk = (cv3 > t_ex) & (taken_f > 0.5)

        def scatter_branch(_):
            plsc.store_scatter(obuf, [iv3], cv3 - t_ex, mask=sup_msk)
            pltpu.async_copy(obuf, out_hbm.at[base + r], osems[r])
            return 0

        def classic_branch(_):
            def op_body(i, _c):
                for k in range(UNROLL):
                    sl = pl.ds((i * UNROLL + k) * LANES, LANES)
                    buf[r, sl] = jnp.maximum(buf[r, sl] - t_ex, 0.0)
                return 0

            lax.fori_loop(0, TRIPS, op_body, 0)
            pltpu.async_copy(buf.at[r], out_hbm.at[base + r], osems[r])
            return 0

        lax.cond(take_scatter, scatter_branch, classic_branch, 0)
        prev_iv = iv3
        prev_msk = sup_msk

    pltpu.make_async_copy(obuf, out_hbm.at[base + RPW - 1],
                          osems[RPW - 1]).wait()


def kernel(logits):
    f = pl.kernel(
        _sparsemax_body,
        out_type=jax.ShapeDtypeStruct((OBS, DIMS), jnp.float32),
        mesh=plsc.VectorSubcoreMesh(core_axis_name="c", subcore_axis_name="s"),
        scratch_types=[
            pltpu.VMEM((RPW, DIMS), jnp.float32),
            pltpu.VMEM((DIMS + LANES,), jnp.int32),
            pltpu.VMEM((DIMS + LANES,), jnp.float32),
            pltpu.VMEM((DIMS + LANES,), jnp.int32),
            pltpu.VMEM((DIMS + LANES,), jnp.float32),
            pltpu.VMEM((DIMS + LANES,), jnp.int32),
            pltpu.VMEM((DIMS,), jnp.float32),
        ] + [pltpu.SemaphoreType.DMA] * (2 * RPW),
        compiler_params=pltpu.CompilerParams(needs_layout_passes=False),
    )
    return f(logits)


# speedup vs baseline: 1.0171x; 1.0171x over previous
import jax
import jax.numpy as jnp
from jax import lax
from jax.experimental import pallas as pl
from jax.experimental.pallas import tpu as pltpu
from jax.experimental.pallas import tpu_sc as plsc

OBS = 128
DIMS = 8192
LANES = 16
CHUNKS = DIMS // LANES
NC = 2
NS = 16
NW = NC * NS
RPW = OBS // NW
BISECT = 30
UNROLL = 16
TRIPS = CHUNKS // UNROLL


def _zeros():
    return jnp.zeros((LANES,), jnp.float32)


def _treemax(cs):
    cs = list(cs)
    while len(cs) > 1:
        cs = [jnp.maximum(cs[j], cs[j + 1]) for j in range(0, len(cs), 2)]
    return cs[0]


def _sparsemax_body(x_hbm, out_hbm, buf, cand, cand2, *sems):
    isems = sems[:RPW]
    osems = sems[RPW:]
    wid = lax.axis_index("s") * NC + lax.axis_index("c")
    base = wid * RPW

    in_copies = [
        pltpu.async_copy(x_hbm.at[base + j], buf.at[j], isems[j])
        for j in range(RPW)
    ]
    out_copies = []

    for r in range(RPW):
        in_copies[r].wait()

        def loadtrip(i):
            return [buf[r, pl.ds(i * (UNROLL * LANES) + k * LANES, LANES)]
                    for k in range(UNROLL)]

        def compact_into(off, cs, msks):
            pcs = [plsc.all_reduce_population_count(m)[0] for m in msks]
            for k in range(len(cs)):
                plsc.store_compressed(cand.at[pl.ds(off, LANES)], cs[k],
                                      mask=msks[k])
                off = off + pcs[k]
            return off

        def pool16(v):
            for sh in (8, 4, 2, 1):
                idx = jnp.bitwise_xor(lax.iota(jnp.int32, LANES), sh)
                v = jnp.maximum(v, v[idx])
            return v

        first = loadtrip(0)
        gm0 = pool16(_treemax(first))
        thr0 = gm0 - 1.0
        cnt0 = compact_into(jnp.int32(0), first, [c > thr0 for c in first])

        def fz_body(i, carry):
            gm, cnt = carry
            cs = loadtrip(i)
            thr = gm - 1.0
            cnt = compact_into(cnt, cs, [c > thr for c in cs])
            return jnp.maximum(gm, pool16(_treemax(cs))), cnt

        gm, cnt = lax.fori_loop(1, TRIPS, fz_body, (gm0, cnt0))
        mv = gm
        lo0 = mv - 1.0
        cand[pl.ds(cnt, LANES)] = lo0

        def s2_body(i, carry):
            c2, kv, sv = carry
            c = cand[pl.ds(i * LANES, LANES)]
            msk = c > lo0
            plsc.store_compressed(cand2.at[pl.ds(c2, LANES)], c, mask=msk)
            return (c2 + plsc.all_reduce_population_count(msk)[0],
                    kv + jnp.where(msk, 1.0, 0.0),
                    sv + jnp.where(msk, c, 0.0))

        nch1 = lax.shift_right_logical(cnt + (LANES - 1), 4)
        cnt2, kv0, sv0 = lax.fori_loop(0, nch1, s2_body,
                                       (jnp.int32(0), _zeros(), _zeros()))
        cand2[pl.ds(cnt2, LANES)] = lo0
        nch2 = lax.shift_right_logical(cnt2 + (LANES - 1), 4)

        tm = jnp.maximum((_zeros() + jnp.sum(sv0) - 1.0) /
                         jnp.maximum(_zeros() + jnp.sum(kv0), 1.0), lo0)

        def mich_round(_, tm):
            def mb(i, carry):
                kv, sv = carry
                c = cand2[pl.ds(i * LANES, LANES)]
                msk = c > tm
                return (kv + jnp.where(msk, 1.0, 0.0),
                        sv + jnp.where(msk, c, 0.0))

            kv, sv = lax.fori_loop(0, nch2, mb, (_zeros(), _zeros()))
            return jnp.maximum(
                (_zeros() + jnp.sum(sv) - 1.0) /
                jnp.maximum(_zeros() + jnp.sum(kv), 1.0), tm)

        tm = lax.fori_loop(0, 2, mich_round, tm)

        def s3_body(i, c3):
            c = cand2[pl.ds(i * LANES, LANES)]
            msk = c > tm
            plsc.store_compressed(cand.at[pl.ds(c3, LANES)], c, mask=msk)
            return c3 + plsc.all_reduce_population_count(msk)[0]

        cnt3 = lax.fori_loop(0, nch2, s3_body, jnp.int32(0))
        cand[pl.ds(cnt3, LANES)] = tm

        def vreg_path(_):
            cv = cand[pl.ds(0, LANES)]
            sk, _sv = plsc.sort_key_val(cv, cv, descending=True)
            csum = plsc.cumsum(sk)
            kf = (lax.iota(jnp.int32, LANES) + 1).astype(jnp.float32)
            check = 1.0 + kf * sk > csum
            kz = plsc.all_reduce_population_count(check)
            tau_sum = csum[kz - 1]
            return (tau_sum - 1.0) / kz.astype(jnp.float32)

        def bisect_path(_):
            def bis_body(j, carry):
                lo, hi = carry
                t = (lo + hi) * 0.5

                def ps(i, a):
                    c = cand2[pl.ds(i * LANES, LANES)]
                    return a + jnp.maximum(c - t, 0.0)

                sv = _zeros() + jnp.sum(lax.fori_loop(0, nch2, ps, _zeros()))
                big = sv >= 1.0
                return jnp.where(big, t, lo), jnp.where(big, hi, t)

            lo, hi = lax.fori_loop(0, BISECT, bis_body, (lo0, mv))

            def ex_body(i, carry):
                kv, sv = carry
                c = cand2[pl.ds(i * LANES, LANES)]
                msk = c > hi
                return (kv + jnp.where(msk, 1.0, 0.0),
                        sv + jnp.where(msk, c, 0.0))

            kv, sv = lax.fori_loop(0, nch2, ex_body, (_zeros(), _zeros()))
            ks = jnp.maximum(_zeros() + jnp.sum(kv), 1.0)
            ss = _zeros() + jnp.sum(sv)
            return (ss - 1.0) / ks

        t_ex = lax.cond(cnt3 <= LANES, vreg_path, bisect_path, 0)

        @plsc.parallel_loop(0, TRIPS)
        def op_body(i):
            for k in range(UNROLL):
                sl = pl.ds((i * UNROLL + k) * LANES, LANES)
                buf[r, sl] = jnp.maximum(buf[r, sl] - t_ex, 0.0)

        out_copies.append(
            pltpu.async_copy(buf.at[r], out_hbm.at[base + r], osems[r]))

    for c in out_copies:
        c.wait()


def kernel(logits):
    f = pl.kernel(
        _sparsemax_body,
        out_type=jax.ShapeDtypeStruct((OBS, DIMS), jnp.float32),
        mesh=plsc.VectorSubcoreMesh(core_axis_name="c", subcore_axis_name="s"),
        scratch_types=[
            pltpu.VMEM((RPW, DIMS), jnp.float32),
            pltpu.VMEM((DIMS + LANES,), jnp.float32),
            pltpu.VMEM((DIMS + LANES,), jnp.float32),
        ] + [pltpu.SemaphoreType.DMA] * (2 * RPW),
        compiler_params=pltpu.CompilerParams(needs_layout_passes=False),
    )
    return f(logits)
